# initial kernel scaffold (unmeasured)
import jax
import jax.numpy as jnp
from jax import lax
from jax.experimental import pallas as pl
from jax.experimental.pallas import tpu as pltpu

NZ = 4
BR = 128


def kernel(x, W):
    t, _ = x.shape
    v_loc = W.shape[1]
    v_glob = NZ * v_loc

    logits = jnp.dot(
        x.astype(jnp.bfloat16),
        W.astype(jnp.bfloat16),
        preferred_element_type=jnp.float32,
    ).astype(jnp.bfloat16)

    def body(lg_ref, out_ref, chunks, staging, send_sems, recv_sems, copy_sem):
        my_x = lax.axis_index("x")
        my_y = lax.axis_index("y")
        my_z = lax.axis_index("z")
        left = (my_z - 1) % NZ
        right = (my_z + 1) % NZ

        barrier = pltpu.get_barrier_semaphore()
        for nbr in (left, right):
            pl.semaphore_signal(
                barrier, inc=1,
                device_id=(my_x, my_y, nbr),
                device_id_type=pl.DeviceIdType.MESH,
            )
        pl.semaphore_wait(barrier, 2)

        for h in range(NZ - 1):
            rdma = pltpu.make_async_remote_copy(
                src_ref=lg_ref if h == 0 else chunks.at[h - 1],
                dst_ref=chunks.at[h],
                send_sem=send_sems.at[h],
                recv_sem=recv_sems.at[h],
                device_id=(my_x, my_y, right),
                device_id_type=pl.DeviceIdType.MESH,
            )
            rdma.start()
            rdma.wait()

        n_blk = t // BR

        ms = []
        for b in range(n_blk):
            r0 = b * BR
            mb = jnp.max(
                lg_ref[pl.ds(r0, BR), :].astype(jnp.float32),
                axis=1, keepdims=True)
            for j in range(NZ - 1):
                mb = jnp.maximum(
                    mb,
                    jnp.max(
                        chunks[j, pl.ds(r0, BR), :].astype(jnp.float32),
                        axis=1, keepdims=True))
            ms.append(mb)

        ss = []
        for b in range(n_blk):
            r0 = b * BR
            sb = jnp.sum(
                jnp.exp(lg_ref[pl.ds(r0, BR), :].astype(jnp.float32) - ms[b]),
                axis=1, keepdims=True)
            for j in range(NZ - 1):
                e = jnp.exp(
                    chunks[j, pl.ds(r0, BR), :].astype(jnp.float32) - ms[b])
                sb = sb + jnp.sum(e, axis=1, keepdims=True)
                chunks[j, pl.ds(r0, BR), :] = e.astype(jnp.bfloat16)
            ss.append(sb)

        for j in range(NZ):
            for b in range(n_blk):
                r0 = b * BR
                rb = 1.0 / ss[b]
                if j < NZ - 1:
                    e = chunks[j, pl.ds(r0, BR), :].astype(jnp.float32)
                else:
                    e = jnp.exp(
                        lg_ref[pl.ds(r0, BR), :].astype(jnp.float32) - ms[b])
                staging[pl.ds(r0, BR), :] = e * rb
            origin = jnp.where(j < NZ - 1, (my_z - j - 1) % NZ, my_z)
            copy = pltpu.make_async_copy(
                staging,
                out_ref.at[:, pl.ds(origin * v_loc, v_loc)],
                copy_sem,
            )
            copy.start()
            copy.wait()

    return pl.pallas_call(
        body,
        out_shape=jax.ShapeDtypeStruct((t, v_glob), jnp.float32),
        in_specs=[pl.BlockSpec(memory_space=pltpu.VMEM)],
        out_specs=pl.BlockSpec(memory_space=pltpu.ANY),
        scratch_shapes=[
            pltpu.VMEM((NZ - 1, t, v_loc), jnp.bfloat16),
            pltpu.VMEM((t, v_loc), jnp.float32),
            pltpu.SemaphoreType.DMA((NZ - 1,)),
            pltpu.SemaphoreType.DMA((NZ - 1,)),
            pltpu.SemaphoreType.DMA,
        ],
        compiler_params=pltpu.CompilerParams(collective_id=0),
    )(logits)


# baseline (device time: 385751 ns/iter reference)
import jax
import jax.numpy as jnp
from jax import lax
from jax.experimental import pallas as pl
from jax.experimental.pallas import tpu as pltpu

NZ = 4
BR = 128


def kernel(x, W):
    t, _ = x.shape
    v_loc = W.shape[1]
    v_glob = NZ * v_loc
    n_blk = t // BR

    logits = jnp.dot(
        x.astype(jnp.bfloat16),
        W.astype(jnp.bfloat16),
        preferred_element_type=jnp.float32,
    ).astype(jnp.bfloat16)

    def body(lg_ref, out_ref, chunks, staging, m_ref, s_ref,
             send_sems, recv_sems, copy_sems):
        my_x = lax.axis_index("x")
        my_y = lax.axis_index("y")
        my_z = lax.axis_index("z")
        left = (my_z - 1) % NZ
        right = (my_z + 1) % NZ

        barrier = pltpu.get_barrier_semaphore()
        for nbr in (left, right):
            pl.semaphore_signal(
                barrier, inc=1,
                device_id=(my_x, my_y, nbr),
                device_id_type=pl.DeviceIdType.MESH,
            )
        pl.semaphore_wait(barrier, 2)

        for h in range(NZ - 1):
            rdma = pltpu.make_async_remote_copy(
                src_ref=lg_ref if h == 0 else chunks.at[h - 1],
                dst_ref=chunks.at[h],
                send_sem=send_sems.at[h],
                recv_sem=recv_sems.at[h],
                device_id=(my_x, my_y, right),
                device_id_type=pl.DeviceIdType.MESH,
            )
            rdma.start()
            rdma.wait()

        def phase_a(b, _):
            r0 = b * BR
            mb = jnp.max(
                lg_ref[pl.ds(r0, BR), :].astype(jnp.float32),
                axis=1, keepdims=True)
            for j in range(NZ - 1):
                mb = jnp.maximum(
                    mb,
                    jnp.max(
                        chunks[j, pl.ds(r0, BR), :].astype(jnp.float32),
                        axis=1, keepdims=True))
            m_ref[pl.ds(r0, BR), :] = mb
            return 0

        lax.fori_loop(0, n_blk, phase_a, 0)

        def phase_b(b, _):
            r0 = b * BR
            mb = m_ref[pl.ds(r0, BR), :]
            sb = jnp.sum(
                jnp.exp(lg_ref[pl.ds(r0, BR), :].astype(jnp.float32) - mb),
                axis=1, keepdims=True)
            for j in range(NZ - 1):
                e = jnp.exp(
                    chunks[j, pl.ds(r0, BR), :].astype(jnp.float32) - mb)
                sb = sb + jnp.sum(e, axis=1, keepdims=True)
                chunks[j, pl.ds(r0, BR), :] = e.astype(jnp.bfloat16)
            s_ref[pl.ds(r0, BR), :] = sb
            return 0

        lax.fori_loop(0, n_blk, phase_b, 0)

        copies = {}
        k = 0
        for j in range(NZ):
            origin = jnp.where(j < NZ - 1, (my_z - j - 1) % NZ, my_z)
            for b in range(n_blk):
                slot = k % 2
                if k >= 2:
                    copies[k - 2].wait()
                r0 = b * BR
                rb = 1.0 / s_ref[pl.ds(r0, BR), :]
                if j < NZ - 1:
                    e = chunks[j, pl.ds(r0, BR), :].astype(jnp.float32)
                else:
                    e = jnp.exp(
                        lg_ref[pl.ds(r0, BR), :].astype(jnp.float32)
                        - m_ref[pl.ds(r0, BR), :])
                staging[slot, :, :] = e * rb
                copy = pltpu.make_async_copy(
                    staging.at[slot],
                    out_ref.at[pl.ds(r0, BR), pl.ds(origin * v_loc, v_loc)],
                    copy_sems.at[slot],
                )
                copy.start()
                copies[k] = copy
                k += 1
        copies[k - 2].wait()
        copies[k - 1].wait()

    return pl.pallas_call(
        body,
        out_shape=jax.ShapeDtypeStruct((t, v_glob), jnp.float32),
        in_specs=[pl.BlockSpec(memory_space=pltpu.MemorySpace.VMEM)],
        out_specs=pl.BlockSpec(memory_space=pl.ANY),
        scratch_shapes=[
            pltpu.VMEM((NZ - 1, t, v_loc), jnp.bfloat16),
            pltpu.VMEM((2, BR, v_loc), jnp.float32),
            pltpu.VMEM((t, 1), jnp.float32),
            pltpu.VMEM((t, 1), jnp.float32),
            pltpu.SemaphoreType.DMA((NZ - 1,)),
            pltpu.SemaphoreType.DMA((NZ - 1,)),
            pltpu.SemaphoreType.DMA((2,)),
        ],
        compiler_params=pltpu.CompilerParams(
            collective_id=0,
            vmem_limit_bytes=60 * 1024 * 1024,
        ),
    )(logits)


# device time: 374436 ns/iter; 1.0302x vs baseline; 1.0302x over previous
import jax
import jax.numpy as jnp
from jax import lax
from jax.experimental import pallas as pl
from jax.experimental.pallas import tpu as pltpu

NZ = 4
BR = 128

F32 = jnp.float32
BF16 = jnp.bfloat16


def kernel(x, W):
    t, _ = x.shape
    v_loc = W.shape[1]
    v_glob = NZ * v_loc
    n_blk = t // BR

    logits = jnp.dot(
        x.astype(BF16),
        W.astype(BF16),
        preferred_element_type=F32,
    ).astype(BF16)

    def body(lg_ref, out_ref, chunks, staging, m_ref, s_ref, snap0, snap1,
             sc_ref, send_sems, recv_sems, copy_sems):
        my_x = lax.axis_index("x")
        my_y = lax.axis_index("y")
        my_z = lax.axis_index("z")
        left = (my_z - 1) % NZ
        right = (my_z + 1) % NZ

        barrier = pltpu.get_barrier_semaphore()
        for nbr in (left, right):
            pl.semaphore_signal(
                barrier, inc=1,
                device_id=(my_x, my_y, nbr),
                device_id_type=pl.DeviceIdType.MESH,
            )
        pl.semaphore_wait(barrier, 2)

        def hop(h):
            return pltpu.make_async_remote_copy(
                src_ref=lg_ref if h == 0 else chunks.at[h - 1],
                dst_ref=chunks.at[h],
                send_sem=send_sems.at[h],
                recv_sem=recv_sems.at[h],
                device_id=(my_x, my_y, right),
                device_id_type=pl.DeviceIdType.MESH,
            )

        def tiled(f):
            lax.fori_loop(0, n_blk, lambda b, _: (f(b * BR), 0)[1], 0)

        rdma0 = hop(0)
        rdma0.start()

        def e1(r0):
            rows = pl.ds(r0, BR)
            mb = jnp.max(lg_ref[rows, :].astype(F32), axis=1, keepdims=True)
            m_ref[rows, :] = mb
            s_ref[rows, :] = jnp.sum(
                jnp.exp(lg_ref[rows, :].astype(F32) - mb),
                axis=1, keepdims=True)
        tiled(e1)

        rdma0.wait()

        rdma1 = hop(1)
        rdma1.start()

        def e2(r0):
            rows = pl.ds(r0, BR)
            mold = m_ref[rows, :]
            mnew = jnp.maximum(
                mold,
                jnp.max(chunks[0, rows, :].astype(F32), axis=1, keepdims=True))
            s_ref[rows, :] = s_ref[rows, :] * jnp.exp(mold - mnew)
            m_ref[rows, :] = mnew
        tiled(e2)

        rdma1.wait()

        rdma2 = hop(2)
        rdma2.start()

        def e34(r0):
            rows = pl.ds(r0, BR)
            msnap = m_ref[rows, :]
            snap0[rows, :] = msnap
            e0 = jnp.exp(chunks[0, rows, :].astype(F32) - msnap)
            chunks[0, rows, :] = e0.astype(BF16)
            sb = s_ref[rows, :] + jnp.sum(e0, axis=1, keepdims=True)
            mnew = jnp.maximum(
                msnap,
                jnp.max(chunks[1, rows, :].astype(F32), axis=1, keepdims=True))
            s_ref[rows, :] = sb * jnp.exp(msnap - mnew)
            m_ref[rows, :] = mnew
        tiled(e34)

        rdma2.wait()

        def e5(r0):
            rows = pl.ds(r0, BR)
            msnap = m_ref[rows, :]
            snap1[rows, :] = msnap
            e1v = jnp.exp(chunks[1, rows, :].astype(F32) - msnap)
            chunks[1, rows, :] = e1v.astype(BF16)
            sb = s_ref[rows, :] + jnp.sum(e1v, axis=1, keepdims=True)
            mf = jnp.maximum(
                msnap,
                jnp.max(chunks[2, rows, :].astype(F32), axis=1, keepdims=True))
            sb = sb * jnp.exp(msnap - mf)
            e2v = jnp.exp(chunks[2, rows, :].astype(F32) - mf)
            chunks[2, rows, :] = e2v.astype(BF16)
            s_ref[rows, :] = sb + jnp.sum(e2v, axis=1, keepdims=True)
            m_ref[rows, :] = mf
        tiled(e5)

        mf = m_ref[:, :]
        rs = 1.0 / s_ref[:, :]
        sc_ref[0, :, :] = jnp.exp(snap0[:, :] - mf) * rs
        sc_ref[1, :, :] = jnp.exp(snap1[:, :] - mf) * rs
        sc_ref[2, :, :] = rs

        copies = {}
        k = 0
        for j in range(NZ):
            origin = jnp.where(j < NZ - 1, (my_z - j - 1) % NZ, my_z)
            for b in range(n_blk):
                slot = k % 2
                if k >= 2:
                    copies[k - 2].wait()
                rows = pl.ds(b * BR, BR)
                if j < NZ - 1:
                    val = chunks[j, rows, :].astype(F32) * sc_ref[j, rows, :]
                else:
                    val = jnp.exp(
                        lg_ref[rows, :].astype(F32) - m_ref[rows, :]
                    ) * (1.0 / s_ref[rows, :])
                staging[slot, :, :] = val
                copy = pltpu.make_async_copy(
                    staging.at[slot],
                    out_ref.at[rows, pl.ds(origin * v_loc, v_loc)],
                    copy_sems.at[slot],
                )
                copy.start()
                copies[k] = copy
                k += 1
        copies[k - 2].wait()
        copies[k - 1].wait()

    return pl.pallas_call(
        body,
        out_shape=jax.ShapeDtypeStruct((t, v_glob), F32),
        in_specs=[pl.BlockSpec(memory_space=pltpu.MemorySpace.VMEM)],
        out_specs=pl.BlockSpec(memory_space=pl.ANY),
        scratch_shapes=[
            pltpu.VMEM((NZ - 1, t, v_loc), BF16),
            pltpu.VMEM((2, BR, v_loc), F32),
            pltpu.VMEM((t, 1), F32),
            pltpu.VMEM((t, 1), F32),
            pltpu.VMEM((t, 1), F32),
            pltpu.VMEM((t, 1), F32),
            pltpu.VMEM((NZ - 1, t, 1), F32),
            pltpu.SemaphoreType.DMA((NZ - 1,)),
            pltpu.SemaphoreType.DMA((NZ - 1,)),
            pltpu.SemaphoreType.DMA((2,)),
        ],
        compiler_params=pltpu.CompilerParams(
            collective_id=0,
            vmem_limit_bytes=60 * 1024 * 1024,
        ),
    )(logits)


# device time: 289161 ns/iter; 1.3340x vs baseline; 1.2949x over previous
import jax
import jax.numpy as jnp
from jax import lax
from jax.experimental import pallas as pl
from jax.experimental.pallas import tpu as pltpu

NZ = 4
BR = 128

F32 = jnp.float32
BF16 = jnp.bfloat16


def kernel(x, W):
    t, _ = x.shape
    v_loc = W.shape[1]
    v_glob = NZ * v_loc
    n_blk = t // BR
    hw = v_loc // 2

    logits = jnp.dot(
        x.astype(BF16),
        W.astype(BF16),
        preferred_element_type=F32,
    ).astype(BF16)

    def body(lg_ref, out_ref, chunks, staging, m_ref, s_ref,
             zsend_sems, zrecv_sems, xsend_sems, xrecv_sems, copy_sems):
        my_x = lax.axis_index("x")
        my_y = lax.axis_index("y")
        my_z = lax.axis_index("z")
        left = (my_z - 1) % NZ
        right = (my_z + 1) % NZ
        my_c0 = my_x * hw
        other_c0 = (1 - my_x) * hw

        barrier = pltpu.get_barrier_semaphore()
        for dev in ((my_x, my_y, left), (my_x, my_y, right),
                    (1 - my_x, my_y, my_z)):
            pl.semaphore_signal(
                barrier, inc=1, device_id=dev,
                device_id_type=pl.DeviceIdType.MESH,
            )
        pl.semaphore_wait(barrier, 3)

        def zhop(h):
            src = (lg_ref.at[:, pl.ds(my_c0, hw)] if h == 0
                   else chunks.at[h - 1, :, pl.ds(my_c0, hw)])
            return pltpu.make_async_remote_copy(
                src_ref=src,
                dst_ref=chunks.at[h, :, pl.ds(my_c0, hw)],
                send_sem=zsend_sems.at[h],
                recv_sem=zrecv_sems.at[h],
                device_id=(my_x, my_y, right),
                device_id_type=pl.DeviceIdType.MESH,
            )

        def xpush(h):
            return pltpu.make_async_remote_copy(
                src_ref=chunks.at[h, :, pl.ds(my_c0, hw)],
                dst_ref=chunks.at[h, :, pl.ds(my_c0, hw)],
                send_sem=xsend_sems.at[h],
                recv_sem=xrecv_sems.at[h],
                device_id=(1 - my_x, my_y, my_z),
                device_id_type=pl.DeviceIdType.MESH,
            )

        def xrecv(h):
            return pltpu.make_async_remote_copy(
                src_ref=chunks.at[h, :, pl.ds(other_c0, hw)],
                dst_ref=chunks.at[h, :, pl.ds(other_c0, hw)],
                send_sem=xsend_sems.at[h],
                recv_sem=xrecv_sems.at[h],
                device_id=(1 - my_x, my_y, my_z),
                device_id_type=pl.DeviceIdType.MESH,
            )

        def tiled(f):
            lax.fori_loop(0, n_blk, lambda b, _: (f(b * BR), 0)[1], 0)

        def fold_own():
            def f(r0):
                rows = pl.ds(r0, BR)
                v = lg_ref[rows, :].astype(F32)
                mb = jnp.max(v, axis=1, keepdims=True)
                m_ref[rows, :] = mb
                s_ref[rows, :] = jnp.sum(
                    jnp.exp(v - mb), axis=1, keepdims=True)
            tiled(f)

        def fold_cols(j, cs):
            def f(r0):
                rows = pl.ds(r0, BR)
                v = chunks[j, rows, cs:cs + hw].astype(F32)
                mold = m_ref[rows, :]
                mnew = jnp.maximum(
                    mold, jnp.max(v, axis=1, keepdims=True))
                s_ref[rows, :] = (
                    s_ref[rows, :] * jnp.exp(mold - mnew)
                    + jnp.sum(jnp.exp(v - mnew), axis=1, keepdims=True))
                m_ref[rows, :] = mnew
            tiled(f)

        def fold_half(j, mine):
            for xs in (0, 1):
                cs = (xs if mine else 1 - xs) * hw

                @pl.when(my_x == xs)
                def _(j=j, cs=cs):
                    fold_cols(j, cs)

        z0 = zhop(0)
        z0.start()
        fold_own()
        z0.wait()

        z1 = zhop(1)
        z1.start()
        x0 = xpush(0)
        x0.start()
        fold_half(0, mine=True)
        xrecv(0).wait_recv()
        fold_half(0, mine=False)
        z1.wait()

        z2 = zhop(2)
        z2.start()
        x1 = xpush(1)
        x1.start()
        fold_half(1, mine=True)
        xrecv(1).wait_recv()
        fold_half(1, mine=False)
        z2.wait()

        x2 = xpush(2)
        x2.start()
        fold_half(2, mine=True)
        xrecv(2).wait_recv()
        fold_half(2, mine=False)

        x0.wait_send()
        x1.wait_send()
        x2.wait_send()

        copies = {}
        k = 0
        for j in range(NZ):
            origin = jnp.where(j < NZ - 1, (my_z - j - 1) % NZ, my_z)
            for b in range(n_blk):
                slot = k % 2
                if k >= 2:
                    copies[k - 2].wait()
                rows = pl.ds(b * BR, BR)
                raw = (chunks[j, rows, :] if j < NZ - 1
                       else lg_ref[rows, :]).astype(F32)
                val = jnp.exp(raw - m_ref[rows, :]) * (1.0 / s_ref[rows, :])
                staging[slot, :, :] = val
                copy = pltpu.make_async_copy(
                    staging.at[slot],
                    out_ref.at[rows, pl.ds(origin * v_loc, v_loc)],
                    copy_sems.at[slot],
                )
                copy.start()
                copies[k] = copy
                k += 1
        copies[k - 2].wait()
        copies[k - 1].wait()

    return pl.pallas_call(
        body,
        out_shape=jax.ShapeDtypeStruct((t, v_glob), F32),
        in_specs=[pl.BlockSpec(memory_space=pltpu.MemorySpace.VMEM)],
        out_specs=pl.BlockSpec(memory_space=pl.ANY),
        scratch_shapes=[
            pltpu.VMEM((NZ - 1, t, v_loc), BF16),
            pltpu.VMEM((2, BR, v_loc), F32),
            pltpu.VMEM((t, 1), F32),
            pltpu.VMEM((t, 1), F32),
            pltpu.SemaphoreType.DMA((NZ - 1,)),
            pltpu.SemaphoreType.DMA((NZ - 1,)),
            pltpu.SemaphoreType.DMA((NZ - 1,)),
            pltpu.SemaphoreType.DMA((NZ - 1,)),
            pltpu.SemaphoreType.DMA((2,)),
        ],
        compiler_params=pltpu.CompilerParams(
            collective_id=0,
            vmem_limit_bytes=60 * 1024 * 1024,
        ),
    )(logits)
